# baseline (speedup 1.0000x reference)
import functools

import jax
import jax.numpy as jnp
from jax import lax
from jax.experimental import pallas as pl
from jax.experimental.pallas import tpu as pltpu
from jax.experimental.pallas import tpu_sc as plsc

EMB_DIM = 128
NUM_ROWS = 64

_mesh = plsc.ScalarSubcoreMesh(axis_name="c", num_cores=1)


@functools.partial(
    pl.kernel,
    mesh=_mesh,
    out_type=jax.ShapeDtypeStruct((NUM_ROWS, EMB_DIM), jnp.float32),
    scratch_types=[
        pltpu.SMEM((NUM_ROWS,), jnp.int32),
        pltpu.SemaphoreType.DMA,
    ],
)
def _gather_kernel(idx_hbm, table_hbm, out_hbm, idx_s, sem):
    pltpu.sync_copy(idx_hbm, idx_s)

    for i in range(NUM_ROWS):
        pltpu.async_copy(table_hbm.at[idx_s[i]], out_hbm.at[i], sem)

    pltpu.make_async_copy(out_hbm, out_hbm, sem).wait()


def kernel(board_idx, piece_embedding):
    idx = board_idx.reshape(NUM_ROWS).astype(jnp.int32)
    return _gather_kernel(idx, piece_embedding)


# speedup vs baseline: 1.0236x; 1.0236x over previous
import functools

import jax
import jax.numpy as jnp
from jax import lax
from jax.experimental import pallas as pl
from jax.experimental.pallas import tpu as pltpu
from jax.experimental.pallas import tpu_sc as plsc

EMB_DIM = 128
NUM_ROWS = 64

_mesh = plsc.ScalarSubcoreMesh(axis_name="c", num_cores=1)


@functools.partial(
    pl.kernel,
    mesh=_mesh,
    out_type=jax.ShapeDtypeStruct((NUM_ROWS, EMB_DIM), jnp.float32),
    scratch_types=[
        pltpu.SMEM((NUM_ROWS,), jnp.int32),
        pltpu.SemaphoreType.DMA,
    ],
)
def _gather_kernel(idx_hbm, table_hbm, out_hbm, idx_s, sem):
    pltpu.sync_copy(idx_hbm, idx_s)

    def issue(i, carry):
        pltpu.async_copy(table_hbm.at[idx_s[i]], out_hbm.at[i], sem)
        return carry

    lax.fori_loop(0, NUM_ROWS, issue, 0)

    pltpu.make_async_copy(out_hbm, out_hbm, sem).wait()


def kernel(board_idx, piece_embedding):
    idx = board_idx.reshape(NUM_ROWS).astype(jnp.int32)
    return _gather_kernel(idx, piece_embedding)
